# baseline (speedup 1.0000x reference)
import functools

import jax
import jax.numpy as jnp
from jax import lax
from jax.experimental import pallas as pl
from jax.experimental.pallas import tpu as pltpu
from jax.experimental.pallas import tpu_sc as plsc

NC, NS, L = 2, 16, 16
NW = NC * NS
B = 16384
D = 64
NE = 1_000_000
NR = 1000
BPW = B // NW
G = D // L
RPB = 16
CHK = 256
NPASS = BPW // CHK


def _make_packer(tcw):
    hs = tcw // 2

    def _body(t_ref, o_ref):
        x = t_ref[...]
        eye = jnp.eye(D, dtype=jnp.float32)
        dn = (((0,), (0,)), ((), ()))
        o_ref[:, 0:D] = lax.dot_general(
            x[:, 0:hs], eye, dn, preferred_element_type=jnp.float32)
        o_ref[:, D:2 * D] = lax.dot_general(
            x[:, hs:tcw], eye, dn, preferred_element_type=jnp.float32)
    return _body


def _to_packed(table_t, n, tcw):
    grid = (n + tcw - 1) // tcw
    return pl.pallas_call(
        _make_packer(tcw),
        grid=(grid,),
        in_specs=[pl.BlockSpec((D, tcw), lambda c: (0, c))],
        out_specs=pl.BlockSpec((tcw // 2, 2 * D), lambda c: (c, 0)),
        out_shape=jax.ShapeDtypeStruct((grid * (tcw // 2), 2 * D), jnp.float32),
    )(table_t)


TCW_E = 32768
TCW_R = 1024
SH_E, SH_R = 14, 9


def _pack_row(v, sh):
    return jnp.bitwise_or(
        lax.shift_left(lax.shift_right_logical(v, sh + 1), sh),
        jnp.bitwise_and(v, (1 << sh) - 1))


def _pack_off(v, sh):
    return jnp.bitwise_and(lax.shift_right_logical(v, sh), 1) * D


_mesh = plsc.VectorSubcoreMesh(core_axis_name="c", subcore_axis_name="s")


@functools.partial(
    pl.kernel,
    out_type=jax.ShapeDtypeStruct((B,), jnp.float32),
    mesh=_mesh,
    scratch_types=[
        pltpu.VMEM((BPW,), jnp.int32),
        pltpu.VMEM((BPW,), jnp.int32),
        pltpu.VMEM((BPW,), jnp.int32),
        pltpu.VMEM((CHK, 2 * D), jnp.float32),
        pltpu.VMEM((CHK, 2 * D), jnp.float32),
        pltpu.VMEM((CHK, 2 * D), jnp.float32),
        pltpu.VMEM((BPW,), jnp.float32),
        pltpu.SemaphoreType.DMA,
    ],
)
def _transe(head_h, rel_h, tail_h, ent_h, relemb_h, out_h,
            ih_v, ir_v, it_v, h_v, r_v, t_v, o_v, sem):
    wid = lax.axis_index("s") * NC + lax.axis_index("c")

    pltpu.sync_copy(head_h.at[wid], ih_v)
    pltpu.sync_copy(rel_h.at[wid], ir_v)
    pltpu.sync_copy(tail_h.at[wid], it_v)

    iot = lax.iota(jnp.int32, L)
    _dnums = lax.GatherDimensionNumbers(
        offset_dims=(), collapsed_slice_dims=(0,), start_index_map=(0,))

    def _perm(v, idx):
        return lax.gather(v, idx.reshape(L, 1), _dnums, (1,),
                          mode=lax.GatherScatterMode.PROMISE_IN_BOUNDS)

    def _hsum(v):
        for s in (8, 4, 2, 1):
            v = v + _perm(v, jnp.bitwise_xor(iot, s))
        return v

    for p in range(NPASS):
        pbase = p * CHK

        def fire(g, c, pbase=pbase):
            base = pbase + g * L
            ihv = ih_v[pl.ds(base, L)]
            irv = ir_v[pl.ds(base, L)]
            itv = it_v[pl.ds(base, L)]
            ihr = _pack_row(ihv, SH_E)
            irr = _pack_row(irv, SH_R)
            itr = _pack_row(itv, SH_E)
            for rr in range(L):
                i = g * L + rr
                pltpu.async_copy(ent_h.at[ihr[rr]], h_v.at[i], sem)
                pltpu.async_copy(relemb_h.at[irr[rr]], r_v.at[i], sem)
                pltpu.async_copy(ent_h.at[itr[rr]], t_v.at[i], sem)
            return c

        lax.fori_loop(0, CHK // L, fire, 0)
        pltpu.make_async_copy(ent_h.at[pl.ds(0, CHK)], h_v, sem).wait()
        pltpu.make_async_copy(ent_h.at[pl.ds(0, CHK)], r_v, sem).wait()
        pltpu.make_async_copy(ent_h.at[pl.ds(0, CHK)], t_v, sem).wait()

        def blk_body(blk, carry, pbase=pbase):
            rbase = blk * RPB
            ihv = ih_v[pl.ds(pbase + rbase, L)]
            irv = ir_v[pl.ds(pbase + rbase, L)]
            itv = it_v[pl.ds(pbase + rbase, L)]
            oh = _pack_off(ihv, SH_E)
            orr = _pack_off(irv, SH_R)
            ot = _pack_off(itv, SH_E)
            outv = jnp.zeros((L,), jnp.float32)
            for rr in range(RPB):
                row = rbase + rr
                ph, pr, pt = oh[rr], orr[rr], ot[rr]
                acc = jnp.zeros((L,), jnp.float32)
                for g in range(G):
                    acc = acc + jnp.abs(
                        h_v[row, pl.ds(ph + g * L, L)]
                        + r_v[row, pl.ds(pr + g * L, L)]
                        - t_v[row, pl.ds(pt + g * L, L)])
                outv = jnp.where(iot == rr, _hsum(acc), outv)
            o_v[pl.ds(pbase + rbase, RPB)] = -outv
            return carry

        lax.fori_loop(0, CHK // RPB, blk_body, 0)

    pltpu.sync_copy(o_v, out_h.at[pl.ds(wid * BPW, BPW)])


def kernel(head, relation, tail, entity_emb, relation_emb):
    head2 = head.astype(jnp.int32).reshape(NW, BPW)
    rel2 = relation.astype(jnp.int32).reshape(NW, BPW)
    tail2 = tail.astype(jnp.int32).reshape(NW, BPW)
    ent_pk = _to_packed(entity_emb.T, NE, TCW_E)
    rel_pk = _to_packed(relation_emb.T, NR, TCW_R)
    return _transe(head2, rel2, tail2, ent_pk, rel_pk)


# speedup vs baseline: 1.0023x; 1.0023x over previous
import functools

import jax
import jax.numpy as jnp
from jax import lax
from jax.experimental import pallas as pl
from jax.experimental.pallas import tpu as pltpu
from jax.experimental.pallas import tpu_sc as plsc

NC, NS, L = 2, 16, 16
NW = NC * NS
B = 16384
D = 64
NE = 1_000_000
NR = 1000
BPW = B // NW
G = D // L
RPB = 16
CHK = 256
NPASS = BPW // CHK


def _make_packer(tcw):
    hs = tcw // 2

    def _body(t_ref, o_ref):
        x = t_ref[...]
        o_ref[:, 0:D] = x[:, 0:hs].T
        o_ref[:, D:2 * D] = x[:, hs:tcw].T
    return _body


def _to_packed(table_t, n, tcw):
    grid = (n + tcw - 1) // tcw
    return pl.pallas_call(
        _make_packer(tcw),
        grid=(grid,),
        in_specs=[pl.BlockSpec((D, tcw), lambda c: (0, c))],
        out_specs=pl.BlockSpec((tcw // 2, 2 * D), lambda c: (c, 0)),
        out_shape=jax.ShapeDtypeStruct((grid * (tcw // 2), 2 * D), jnp.float32),
    )(table_t)


TCW_E = 32768
TCW_R = 1024
SH_E, SH_R = 14, 9


def _pack_row(v, sh):
    return jnp.bitwise_or(
        lax.shift_left(lax.shift_right_logical(v, sh + 1), sh),
        jnp.bitwise_and(v, (1 << sh) - 1))


def _pack_off(v, sh):
    return jnp.bitwise_and(lax.shift_right_logical(v, sh), 1) * D


_mesh = plsc.VectorSubcoreMesh(core_axis_name="c", subcore_axis_name="s")


@functools.partial(
    pl.kernel,
    out_type=jax.ShapeDtypeStruct((B,), jnp.float32),
    mesh=_mesh,
    scratch_types=[
        pltpu.VMEM((BPW,), jnp.int32),
        pltpu.VMEM((BPW,), jnp.int32),
        pltpu.VMEM((BPW,), jnp.int32),
        pltpu.VMEM((CHK, 2 * D), jnp.float32),
        pltpu.VMEM((CHK, 2 * D), jnp.float32),
        pltpu.VMEM((CHK, 2 * D), jnp.float32),
        pltpu.VMEM((BPW,), jnp.float32),
        pltpu.SemaphoreType.DMA,
    ],
)
def _transe(head_h, rel_h, tail_h, ent_h, relemb_h, out_h,
            ih_v, ir_v, it_v, h_v, r_v, t_v, o_v, sem):
    wid = lax.axis_index("s") * NC + lax.axis_index("c")

    pltpu.sync_copy(head_h.at[wid], ih_v)
    pltpu.sync_copy(rel_h.at[wid], ir_v)
    pltpu.sync_copy(tail_h.at[wid], it_v)

    iot = lax.iota(jnp.int32, L)
    _dnums = lax.GatherDimensionNumbers(
        offset_dims=(), collapsed_slice_dims=(0,), start_index_map=(0,))

    def _perm(v, idx):
        return lax.gather(v, idx.reshape(L, 1), _dnums, (1,),
                          mode=lax.GatherScatterMode.PROMISE_IN_BOUNDS)

    def _hsum(v):
        for s in (8, 4, 2, 1):
            v = v + _perm(v, jnp.bitwise_xor(iot, s))
        return v

    for p in range(NPASS):
        pbase = p * CHK

        def fire(g, c, pbase=pbase):
            base = pbase + g * L
            ihv = ih_v[pl.ds(base, L)]
            irv = ir_v[pl.ds(base, L)]
            itv = it_v[pl.ds(base, L)]
            ihr = _pack_row(ihv, SH_E)
            irr = _pack_row(irv, SH_R)
            itr = _pack_row(itv, SH_E)
            for rr in range(L):
                i = g * L + rr
                pltpu.async_copy(ent_h.at[ihr[rr]], h_v.at[i], sem)
                pltpu.async_copy(relemb_h.at[irr[rr]], r_v.at[i], sem)
                pltpu.async_copy(ent_h.at[itr[rr]], t_v.at[i], sem)
            return c

        lax.fori_loop(0, CHK // L, fire, 0)
        pltpu.make_async_copy(ent_h.at[pl.ds(0, CHK)], h_v, sem).wait()
        pltpu.make_async_copy(ent_h.at[pl.ds(0, CHK)], r_v, sem).wait()
        pltpu.make_async_copy(ent_h.at[pl.ds(0, CHK)], t_v, sem).wait()

        def blk_body(blk, carry, pbase=pbase):
            rbase = blk * RPB
            ihv = ih_v[pl.ds(pbase + rbase, L)]
            irv = ir_v[pl.ds(pbase + rbase, L)]
            itv = it_v[pl.ds(pbase + rbase, L)]
            oh = _pack_off(ihv, SH_E)
            orr = _pack_off(irv, SH_R)
            ot = _pack_off(itv, SH_E)
            outv = jnp.zeros((L,), jnp.float32)
            for rr in range(RPB):
                row = rbase + rr
                ph, pr, pt = oh[rr], orr[rr], ot[rr]
                acc = jnp.zeros((L,), jnp.float32)
                for g in range(G):
                    acc = acc + jnp.abs(
                        h_v[row, pl.ds(ph + g * L, L)]
                        + r_v[row, pl.ds(pr + g * L, L)]
                        - t_v[row, pl.ds(pt + g * L, L)])
                outv = jnp.where(iot == rr, _hsum(acc), outv)
            o_v[pl.ds(pbase + rbase, RPB)] = -outv
            return carry

        lax.fori_loop(0, CHK // RPB, blk_body, 0)

    pltpu.sync_copy(o_v, out_h.at[pl.ds(wid * BPW, BPW)])


def kernel(head, relation, tail, entity_emb, relation_emb):
    head2 = head.astype(jnp.int32).reshape(NW, BPW)
    rel2 = relation.astype(jnp.int32).reshape(NW, BPW)
    tail2 = tail.astype(jnp.int32).reshape(NW, BPW)
    ent_pk = _to_packed(entity_emb.T, NE, TCW_E)
    rel_pk = _to_packed(relation_emb.T, NR, TCW_R)
    return _transe(head2, rel2, tail2, ent_pk, rel_pk)


# speedup vs baseline: 1.3737x; 1.3705x over previous
import functools

import jax
import jax.numpy as jnp
from jax import lax
from jax.experimental import pallas as pl
from jax.experimental.pallas import tpu as pltpu
from jax.experimental.pallas import tpu_sc as plsc

NC, NS, L = 2, 16, 16
NW = NC * NS
B = 16384
D = 64
NE = 1_000_000
NR = 1000
BPW = B // NW
G = D // L
RPB = 16
CHK = 256
NPASS = BPW // CHK


def _make_packer(tcw):
    qs = tcw // 4

    def _bits(y):
        yb = y.astype(jnp.bfloat16)
        return lax.bitcast_convert_type(yb, jnp.uint16).astype(jnp.uint32)

    def _body(t_ref, o_ref):
        x = t_ref[...]
        y = [_bits(x[:, k * qs:(k + 1) * qs].T) for k in range(4)]
        o_ref[:, 0:D] = jnp.bitwise_or(y[0], lax.shift_left(y[1], jnp.uint32(16)))
        o_ref[:, D:2 * D] = jnp.bitwise_or(y[2], lax.shift_left(y[3], jnp.uint32(16)))
    return _body


def _to_packed(table_t, n, tcw):
    grid = (n + tcw - 1) // tcw
    return pl.pallas_call(
        _make_packer(tcw),
        grid=(grid,),
        in_specs=[pl.BlockSpec((D, tcw), lambda c: (0, c))],
        out_specs=pl.BlockSpec((tcw // 4, 2 * D), lambda c: (c, 0)),
        out_shape=jax.ShapeDtypeStruct((grid * (tcw // 4), 2 * D), jnp.uint32),
    )(table_t)


TCW_E = 32768
TCW_R = 1024
SB_E, SQ_E = 15, 13
SB_R, SQ_R = 10, 8


def _pack_row(v, sb, sq):
    return jnp.bitwise_or(
        lax.shift_left(lax.shift_right_logical(v, sb), sq),
        jnp.bitwise_and(v, (1 << sq) - 1))


def _pack_woff(v, sq):
    return jnp.bitwise_and(lax.shift_right_logical(v, sq + 1), 1) * D


def _pack_half(v, sq):
    return jnp.bitwise_and(lax.shift_right_logical(v, sq), 1)


_mesh = plsc.VectorSubcoreMesh(core_axis_name="c", subcore_axis_name="s")


@functools.partial(
    pl.kernel,
    out_type=jax.ShapeDtypeStruct((B,), jnp.float32),
    mesh=_mesh,
    scratch_types=[
        pltpu.VMEM((BPW,), jnp.int32),
        pltpu.VMEM((BPW,), jnp.int32),
        pltpu.VMEM((BPW,), jnp.int32),
        pltpu.VMEM((CHK, 2 * D), jnp.uint32),
        pltpu.VMEM((CHK, 2 * D), jnp.uint32),
        pltpu.VMEM((CHK, 2 * D), jnp.uint32),
        pltpu.VMEM((BPW,), jnp.float32),
        pltpu.SemaphoreType.DMA,
    ],
)
def _transe(head_h, rel_h, tail_h, ent_h, relemb_h, out_h,
            ih_v, ir_v, it_v, h_v, r_v, t_v, o_v, sem):
    wid = lax.axis_index("s") * NC + lax.axis_index("c")

    pltpu.sync_copy(head_h.at[wid], ih_v)
    pltpu.sync_copy(rel_h.at[wid], ir_v)
    pltpu.sync_copy(tail_h.at[wid], it_v)

    iot = lax.iota(jnp.int32, L)
    _dnums = lax.GatherDimensionNumbers(
        offset_dims=(), collapsed_slice_dims=(0,), start_index_map=(0,))

    def _perm(v, idx):
        return lax.gather(v, idx.reshape(L, 1), _dnums, (1,),
                          mode=lax.GatherScatterMode.PROMISE_IN_BOUNDS)

    def _hsum(v):
        for s in (8, 4, 2, 1):
            v = v + _perm(v, jnp.bitwise_xor(iot, s))
        return v

    for p in range(NPASS):
        pbase = p * CHK

        def fire(g, c, pbase=pbase):
            base = pbase + g * L
            ihv = ih_v[pl.ds(base, L)]
            irv = ir_v[pl.ds(base, L)]
            itv = it_v[pl.ds(base, L)]
            ihr = _pack_row(ihv, SB_E, SQ_E)
            irr = _pack_row(irv, SB_R, SQ_R)
            itr = _pack_row(itv, SB_E, SQ_E)
            for rr in range(L):
                i = g * L + rr
                pltpu.async_copy(ent_h.at[ihr[rr]], h_v.at[i], sem)
                pltpu.async_copy(relemb_h.at[irr[rr]], r_v.at[i], sem)
                pltpu.async_copy(ent_h.at[itr[rr]], t_v.at[i], sem)
            return c

        lax.fori_loop(0, CHK // L, fire, 0)
        pltpu.make_async_copy(ent_h.at[pl.ds(0, CHK)], h_v, sem).wait()
        pltpu.make_async_copy(ent_h.at[pl.ds(0, CHK)], r_v, sem).wait()
        pltpu.make_async_copy(ent_h.at[pl.ds(0, CHK)], t_v, sem).wait()

        def blk_body(blk, carry, pbase=pbase):
            rbase = blk * RPB
            ihv = ih_v[pl.ds(pbase + rbase, L)]
            irv = ir_v[pl.ds(pbase + rbase, L)]
            itv = it_v[pl.ds(pbase + rbase, L)]
            oh = _pack_woff(ihv, SQ_E)
            orr = _pack_woff(irv, SQ_R)
            ot = _pack_woff(itv, SQ_E)
            hh = _pack_half(ihv, SQ_E)
            hr = _pack_half(irv, SQ_R)
            ht = _pack_half(itv, SQ_E)

            def _feat(ref, row, off, shv, g):
                w = ref[row, pl.ds(off + g * L, L)]
                bits = lax.shift_left(lax.shift_right_logical(w, shv),
                                      jnp.uint32(16))
                return lax.bitcast_convert_type(bits, jnp.float32)

            outv = jnp.zeros((L,), jnp.float32)
            for rr in range(RPB):
                row = rbase + rr
                ph, pr, pt = oh[rr], orr[rr], ot[rr]
                ch = jnp.zeros((L,), jnp.uint32) + (hh[rr] * 16).astype(jnp.uint32)
                cr = jnp.zeros((L,), jnp.uint32) + (hr[rr] * 16).astype(jnp.uint32)
                ct = jnp.zeros((L,), jnp.uint32) + (ht[rr] * 16).astype(jnp.uint32)
                acc = jnp.zeros((L,), jnp.float32)
                for g in range(G):
                    acc = acc + jnp.abs(
                        _feat(h_v, row, ph, ch, g)
                        + _feat(r_v, row, pr, cr, g)
                        - _feat(t_v, row, pt, ct, g))
                outv = jnp.where(iot == rr, _hsum(acc), outv)
            o_v[pl.ds(pbase + rbase, RPB)] = -outv
            return carry

        lax.fori_loop(0, CHK // RPB, blk_body, 0)

    pltpu.sync_copy(o_v, out_h.at[pl.ds(wid * BPW, BPW)])


def kernel(head, relation, tail, entity_emb, relation_emb):
    head2 = head.astype(jnp.int32).reshape(NW, BPW)
    rel2 = relation.astype(jnp.int32).reshape(NW, BPW)
    tail2 = tail.astype(jnp.int32).reshape(NW, BPW)
    ent_pk = _to_packed(entity_emb.T, NE, TCW_E)
    rel_pk = _to_packed(relation_emb.T, NR, TCW_R)
    return _transe(head2, rel2, tail2, ent_pk, rel_pk)


# speedup vs baseline: 1.4158x; 1.0306x over previous
import functools

import jax
import jax.numpy as jnp
from jax import lax
from jax.experimental import pallas as pl
from jax.experimental.pallas import tpu as pltpu
from jax.experimental.pallas import tpu_sc as plsc

NC, NS, L = 2, 16, 16
NW = NC * NS
B = 16384
D = 64
NE = 1_000_000
NR = 1000
BPW = B // NW
G = D // L
RPB = 16
CHK = 256
NPASS = BPW // CHK


def _make_packer(tcw):
    qs = tcw // 4

    def _bits(y):
        yb = y.astype(jnp.bfloat16)
        return lax.bitcast_convert_type(yb, jnp.uint16).astype(jnp.uint32)

    def _body(t_ref, o_ref):
        x = t_ref[...]
        y = [_bits(x[:, k * qs:(k + 1) * qs].T) for k in range(4)]
        o_ref[:, 0:D] = jnp.bitwise_or(y[0], lax.shift_left(y[1], jnp.uint32(16)))
        o_ref[:, D:2 * D] = jnp.bitwise_or(y[2], lax.shift_left(y[3], jnp.uint32(16)))
    return _body


def _to_packed(table_t, n, tcw):
    grid = (n + tcw - 1) // tcw
    return pl.pallas_call(
        _make_packer(tcw),
        grid=(grid,),
        in_specs=[pl.BlockSpec((D, tcw), lambda c: (0, c))],
        out_specs=pl.BlockSpec((tcw // 4, 2 * D), lambda c: (c, 0)),
        out_shape=jax.ShapeDtypeStruct((grid * (tcw // 4), 2 * D), jnp.uint32),
    )(table_t)


TCW_E = 32768
TCW_R = 1024
SB_E, SQ_E = 15, 13
SB_R, SQ_R = 10, 8


def _pack_row(v, sb, sq):
    return jnp.bitwise_or(
        lax.shift_left(lax.shift_right_logical(v, sb), sq),
        jnp.bitwise_and(v, (1 << sq) - 1))


def _pack_woff(v, sq):
    return jnp.bitwise_and(lax.shift_right_logical(v, sq + 1), 1) * D


def _pack_half(v, sq):
    return jnp.bitwise_and(lax.shift_right_logical(v, sq), 1)


_mesh = plsc.VectorSubcoreMesh(core_axis_name="c", subcore_axis_name="s")


@functools.partial(
    pl.kernel,
    out_type=jax.ShapeDtypeStruct((B,), jnp.float32),
    mesh=_mesh,
    scratch_types=[
        pltpu.VMEM((BPW,), jnp.int32),
        pltpu.VMEM((BPW,), jnp.int32),
        pltpu.VMEM((BPW,), jnp.int32),
        pltpu.VMEM((CHK, 2 * D), jnp.uint32),
        pltpu.VMEM((CHK, 2 * D), jnp.uint32),
        pltpu.VMEM((CHK, 2 * D), jnp.uint32),
        pltpu.VMEM((BPW,), jnp.float32),
        pltpu.SemaphoreType.DMA,
    ],
)
def _transe(head_h, rel_h, tail_h, ent_h, relemb_h, out_h,
            ih_v, ir_v, it_v, h_v, r_v, t_v, o_v, sem):
    wid = lax.axis_index("s") * NC + lax.axis_index("c")

    pltpu.sync_copy(head_h.at[pl.ds(wid * BPW, BPW)], ih_v)
    pltpu.sync_copy(rel_h.at[pl.ds(wid * BPW, BPW)], ir_v)
    pltpu.sync_copy(tail_h.at[pl.ds(wid * BPW, BPW)], it_v)

    iot = lax.iota(jnp.int32, L)
    _dnums = lax.GatherDimensionNumbers(
        offset_dims=(), collapsed_slice_dims=(0,), start_index_map=(0,))

    def _perm(v, idx):
        return lax.gather(v, idx.reshape(L, 1), _dnums, (1,),
                          mode=lax.GatherScatterMode.PROMISE_IN_BOUNDS)

    def _hsum(v):
        for s in (8, 4, 2, 1):
            v = v + _perm(v, jnp.bitwise_xor(iot, s))
        return v

    for p in range(NPASS):
        pbase = p * CHK

        def fire(g, c, pbase=pbase):
            base = pbase + g * L
            ihv = ih_v[pl.ds(base, L)]
            irv = ir_v[pl.ds(base, L)]
            itv = it_v[pl.ds(base, L)]
            ihr = _pack_row(ihv, SB_E, SQ_E)
            irr = _pack_row(irv, SB_R, SQ_R)
            itr = _pack_row(itv, SB_E, SQ_E)
            for rr in range(L):
                i = g * L + rr
                pltpu.async_copy(ent_h.at[ihr[rr]], h_v.at[i], sem)
                pltpu.async_copy(relemb_h.at[irr[rr]], r_v.at[i], sem)
                pltpu.async_copy(ent_h.at[itr[rr]], t_v.at[i], sem)
            return c

        lax.fori_loop(0, CHK // L, fire, 0)
        pltpu.make_async_copy(ent_h.at[pl.ds(0, CHK)], h_v, sem).wait()
        pltpu.make_async_copy(ent_h.at[pl.ds(0, CHK)], r_v, sem).wait()
        pltpu.make_async_copy(ent_h.at[pl.ds(0, CHK)], t_v, sem).wait()

        def blk_body(blk, carry, pbase=pbase):
            rbase = blk * RPB
            ihv = ih_v[pl.ds(pbase + rbase, L)]
            irv = ir_v[pl.ds(pbase + rbase, L)]
            itv = it_v[pl.ds(pbase + rbase, L)]
            oh = _pack_woff(ihv, SQ_E)
            orr = _pack_woff(irv, SQ_R)
            ot = _pack_woff(itv, SQ_E)
            hh = _pack_half(ihv, SQ_E)
            hr = _pack_half(irv, SQ_R)
            ht = _pack_half(itv, SQ_E)

            def _feat(ref, row, off, shv, g):
                w = ref[row, pl.ds(off + g * L, L)]
                bits = lax.shift_left(lax.shift_right_logical(w, shv),
                                      jnp.uint32(16))
                return lax.bitcast_convert_type(bits, jnp.float32)

            outv = jnp.zeros((L,), jnp.float32)
            for rr in range(RPB):
                row = rbase + rr
                ph, pr, pt = oh[rr], orr[rr], ot[rr]
                ch = jnp.zeros((L,), jnp.uint32) + (hh[rr] * 16).astype(jnp.uint32)
                cr = jnp.zeros((L,), jnp.uint32) + (hr[rr] * 16).astype(jnp.uint32)
                ct = jnp.zeros((L,), jnp.uint32) + (ht[rr] * 16).astype(jnp.uint32)
                acc = jnp.zeros((L,), jnp.float32)
                for g in range(G):
                    acc = acc + jnp.abs(
                        _feat(h_v, row, ph, ch, g)
                        + _feat(r_v, row, pr, cr, g)
                        - _feat(t_v, row, pt, ct, g))
                outv = jnp.where(iot == rr, _hsum(acc), outv)
            o_v[pl.ds(pbase + rbase, RPB)] = -outv
            return carry

        lax.fori_loop(0, CHK // RPB, blk_body, 0)

    pltpu.sync_copy(o_v, out_h.at[pl.ds(wid * BPW, BPW)])


def kernel(head, relation, tail, entity_emb, relation_emb):
    ent_pk = _to_packed(entity_emb.T, NE, TCW_E)
    rel_pk = _to_packed(relation_emb.T, NR, TCW_R)
    return _transe(head.astype(jnp.int32), relation.astype(jnp.int32),
                   tail.astype(jnp.int32), ent_pk, rel_pk)


# speedup vs baseline: 1.4226x; 1.0048x over previous
import functools

import jax
import jax.numpy as jnp
from jax import lax
from jax.experimental import pallas as pl
from jax.experimental.pallas import tpu as pltpu
from jax.experimental.pallas import tpu_sc as plsc

NC, NS, L = 2, 16, 16
NW = NC * NS
B = 16384
D = 64
NE = 1_000_000
NR = 1000
BPW = B // NW
G = D // L
RPB = 16
CHK = 256
NPASS = BPW // CHK


_SUB = 2048


def _make_packer(tcw):
    qs = tcw // 4

    def _bits(y):
        yb = y.astype(jnp.bfloat16)
        return lax.bitcast_convert_type(yb, jnp.uint16).astype(jnp.uint32)

    def _body(t_ref, o_ref):
        sub = min(_SUB, qs)
        for klo, khi, c0 in ((0, 1, 0), (2, 3, D)):
            for m in range(qs // sub):
                rows = pl.ds(m * sub, sub)
                lo = _bits(t_ref[:, pl.ds(klo * qs + m * sub, sub)].T)
                hi = _bits(t_ref[:, pl.ds(khi * qs + m * sub, sub)].T)
                o_ref[rows, c0:c0 + D] = jnp.bitwise_or(
                    lo, lax.shift_left(hi, jnp.uint32(16)))
    return _body


def _to_packed(table_t, n, tcw):
    grid = (n + tcw - 1) // tcw
    return pl.pallas_call(
        _make_packer(tcw),
        grid=(grid,),
        in_specs=[pl.BlockSpec((D, tcw), lambda c: (0, c))],
        out_specs=pl.BlockSpec((tcw // 4, 2 * D), lambda c: (c, 0)),
        out_shape=jax.ShapeDtypeStruct((grid * (tcw // 4), 2 * D), jnp.uint32),
    )(table_t)


TCW_E = 32768
TCW_R = 1024
SB_E, SQ_E = 15, 13
SB_R, SQ_R = 10, 8


def _pack_row(v, sb, sq):
    return jnp.bitwise_or(
        lax.shift_left(lax.shift_right_logical(v, sb), sq),
        jnp.bitwise_and(v, (1 << sq) - 1))


def _pack_woff(v, sq):
    return jnp.bitwise_and(lax.shift_right_logical(v, sq + 1), 1) * D


def _pack_half(v, sq):
    return jnp.bitwise_and(lax.shift_right_logical(v, sq), 1)


_mesh = plsc.VectorSubcoreMesh(core_axis_name="c", subcore_axis_name="s")


@functools.partial(
    pl.kernel,
    out_type=jax.ShapeDtypeStruct((B,), jnp.float32),
    mesh=_mesh,
    scratch_types=[
        pltpu.VMEM((BPW,), jnp.int32),
        pltpu.VMEM((BPW,), jnp.int32),
        pltpu.VMEM((BPW,), jnp.int32),
        pltpu.VMEM((CHK, 2 * D), jnp.uint32),
        pltpu.VMEM((CHK, 2 * D), jnp.uint32),
        pltpu.VMEM((CHK, 2 * D), jnp.uint32),
        pltpu.VMEM((BPW,), jnp.float32),
        pltpu.SemaphoreType.DMA,
    ],
)
def _transe(head_h, rel_h, tail_h, ent_h, relemb_h, out_h,
            ih_v, ir_v, it_v, h_v, r_v, t_v, o_v, sem):
    wid = lax.axis_index("s") * NC + lax.axis_index("c")

    pltpu.sync_copy(head_h.at[pl.ds(wid * BPW, BPW)], ih_v)
    pltpu.sync_copy(rel_h.at[pl.ds(wid * BPW, BPW)], ir_v)
    pltpu.sync_copy(tail_h.at[pl.ds(wid * BPW, BPW)], it_v)

    iot = lax.iota(jnp.int32, L)
    _dnums = lax.GatherDimensionNumbers(
        offset_dims=(), collapsed_slice_dims=(0,), start_index_map=(0,))

    def _perm(v, idx):
        return lax.gather(v, idx.reshape(L, 1), _dnums, (1,),
                          mode=lax.GatherScatterMode.PROMISE_IN_BOUNDS)

    def _hsum(v):
        for s in (8, 4, 2, 1):
            v = v + _perm(v, jnp.bitwise_xor(iot, s))
        return v

    for p in range(NPASS):
        pbase = p * CHK

        def fire(g, c, pbase=pbase):
            base = pbase + g * L
            ihv = ih_v[pl.ds(base, L)]
            irv = ir_v[pl.ds(base, L)]
            itv = it_v[pl.ds(base, L)]
            ihr = _pack_row(ihv, SB_E, SQ_E)
            irr = _pack_row(irv, SB_R, SQ_R)
            itr = _pack_row(itv, SB_E, SQ_E)
            for rr in range(L):
                i = g * L + rr
                pltpu.async_copy(ent_h.at[ihr[rr]], h_v.at[i], sem)
                pltpu.async_copy(relemb_h.at[irr[rr]], r_v.at[i], sem)
                pltpu.async_copy(ent_h.at[itr[rr]], t_v.at[i], sem)
            return c

        lax.fori_loop(0, CHK // L, fire, 0)
        pltpu.make_async_copy(ent_h.at[pl.ds(0, CHK)], h_v, sem).wait()
        pltpu.make_async_copy(ent_h.at[pl.ds(0, CHK)], r_v, sem).wait()
        pltpu.make_async_copy(ent_h.at[pl.ds(0, CHK)], t_v, sem).wait()

        def blk_body(blk, carry, pbase=pbase):
            rbase = blk * RPB
            ihv = ih_v[pl.ds(pbase + rbase, L)]
            irv = ir_v[pl.ds(pbase + rbase, L)]
            itv = it_v[pl.ds(pbase + rbase, L)]
            oh = _pack_woff(ihv, SQ_E)
            orr = _pack_woff(irv, SQ_R)
            ot = _pack_woff(itv, SQ_E)
            hh = _pack_half(ihv, SQ_E)
            hr = _pack_half(irv, SQ_R)
            ht = _pack_half(itv, SQ_E)

            def _feat(ref, row, off, shv, g):
                w = ref[row, pl.ds(off + g * L, L)]
                bits = lax.shift_left(lax.shift_right_logical(w, shv),
                                      jnp.uint32(16))
                return lax.bitcast_convert_type(bits, jnp.float32)

            outv = jnp.zeros((L,), jnp.float32)
            for rr in range(RPB):
                row = rbase + rr
                ph, pr, pt = oh[rr], orr[rr], ot[rr]
                ch = jnp.zeros((L,), jnp.uint32) + (hh[rr] * 16).astype(jnp.uint32)
                cr = jnp.zeros((L,), jnp.uint32) + (hr[rr] * 16).astype(jnp.uint32)
                ct = jnp.zeros((L,), jnp.uint32) + (ht[rr] * 16).astype(jnp.uint32)
                acc = jnp.zeros((L,), jnp.float32)
                for g in range(G):
                    acc = acc + jnp.abs(
                        _feat(h_v, row, ph, ch, g)
                        + _feat(r_v, row, pr, cr, g)
                        - _feat(t_v, row, pt, ct, g))
                outv = jnp.where(iot == rr, _hsum(acc), outv)
            o_v[pl.ds(pbase + rbase, RPB)] = -outv
            return carry

        lax.fori_loop(0, CHK // RPB, blk_body, 0)

    pltpu.sync_copy(o_v, out_h.at[pl.ds(wid * BPW, BPW)])


def kernel(head, relation, tail, entity_emb, relation_emb):
    ent_pk = _to_packed(entity_emb.T, NE, TCW_E)
    rel_pk = _to_packed(relation_emb.T, NR, TCW_R)
    return _transe(head.astype(jnp.int32), relation.astype(jnp.int32),
                   tail.astype(jnp.int32), ent_pk, rel_pk)


# speedup vs baseline: 1.4482x; 1.0180x over previous
import functools

import jax
import jax.numpy as jnp
from jax import lax
from jax.experimental import pallas as pl
from jax.experimental.pallas import tpu as pltpu
from jax.experimental.pallas import tpu_sc as plsc

NC, NS, L = 2, 16, 16
NW = NC * NS
B = 16384
D = 64
NE = 1_000_000
NR = 1000
BPW = B // NW
G = D // L
RPB = 16
CHK = 128
NPASS = BPW // CHK


_SUB = 2048


def _make_packer(tcw):
    qs = tcw // 4

    def _bits(y):
        yb = y.astype(jnp.bfloat16)
        return lax.bitcast_convert_type(yb, jnp.uint16).astype(jnp.uint32)

    def _body(t_ref, o_ref):
        sub = min(_SUB, qs)
        for klo, khi, c0 in ((0, 1, 0), (2, 3, D)):
            for m in range(qs // sub):
                rows = pl.ds(m * sub, sub)
                lo = _bits(t_ref[:, pl.ds(klo * qs + m * sub, sub)].T)
                hi = _bits(t_ref[:, pl.ds(khi * qs + m * sub, sub)].T)
                o_ref[rows, c0:c0 + D] = jnp.bitwise_or(
                    lo, lax.shift_left(hi, jnp.uint32(16)))
    return _body


def _to_packed(table_t, n, tcw):
    grid = (n + tcw - 1) // tcw
    return pl.pallas_call(
        _make_packer(tcw),
        grid=(grid,),
        in_specs=[pl.BlockSpec((D, tcw), lambda c: (0, c))],
        out_specs=pl.BlockSpec((tcw // 4, 2 * D), lambda c: (c, 0)),
        out_shape=jax.ShapeDtypeStruct((grid * (tcw // 4), 2 * D), jnp.uint32),
    )(table_t)


TCW_E = 32768
TCW_R = 1024
SB_E, SQ_E = 15, 13
SB_R, SQ_R = 10, 8


def _pack_row(v, sb, sq):
    return jnp.bitwise_or(
        lax.shift_left(lax.shift_right_logical(v, sb), sq),
        jnp.bitwise_and(v, (1 << sq) - 1))


def _pack_woff(v, sq):
    return jnp.bitwise_and(lax.shift_right_logical(v, sq + 1), 1) * D


def _pack_half(v, sq):
    return jnp.bitwise_and(lax.shift_right_logical(v, sq), 1)


_mesh = plsc.VectorSubcoreMesh(core_axis_name="c", subcore_axis_name="s")


@functools.partial(
    pl.kernel,
    out_type=jax.ShapeDtypeStruct((B,), jnp.float32),
    mesh=_mesh,
    scratch_types=[
        pltpu.VMEM((BPW,), jnp.int32),
        pltpu.VMEM((BPW,), jnp.int32),
        pltpu.VMEM((BPW,), jnp.int32),
        pltpu.VMEM((2, CHK, 2 * D), jnp.uint32),
        pltpu.VMEM((2, CHK, 2 * D), jnp.uint32),
        pltpu.VMEM((2, CHK, 2 * D), jnp.uint32),
        pltpu.VMEM((BPW,), jnp.float32),
        pltpu.SemaphoreType.DMA,
        pltpu.SemaphoreType.DMA,
    ],
)
def _transe(head_h, rel_h, tail_h, ent_h, relemb_h, out_h,
            ih_v, ir_v, it_v, h_v, r_v, t_v, o_v, sem0, sem1):
    wid = lax.axis_index("s") * NC + lax.axis_index("c")

    pltpu.sync_copy(head_h.at[pl.ds(wid * BPW, BPW)], ih_v)
    pltpu.sync_copy(rel_h.at[pl.ds(wid * BPW, BPW)], ir_v)
    pltpu.sync_copy(tail_h.at[pl.ds(wid * BPW, BPW)], it_v)

    iot = lax.iota(jnp.int32, L)
    _dnums = lax.GatherDimensionNumbers(
        offset_dims=(), collapsed_slice_dims=(0,), start_index_map=(0,))

    def _perm(v, idx):
        return lax.gather(v, idx.reshape(L, 1), _dnums, (1,),
                          mode=lax.GatherScatterMode.PROMISE_IN_BOUNDS)

    def _hsum(v):
        for s in (8, 4, 2, 1):
            v = v + _perm(v, jnp.bitwise_xor(iot, s))
        return v

    sems = (sem0, sem1)

    def _fire_pass(p):
        b, sem = p % 2, sems[p % 2]

        def fire(g, c):
            base = p * CHK + g * L
            ihv = ih_v[pl.ds(base, L)]
            irv = ir_v[pl.ds(base, L)]
            itv = it_v[pl.ds(base, L)]
            ihr = _pack_row(ihv, SB_E, SQ_E)
            irr = _pack_row(irv, SB_R, SQ_R)
            itr = _pack_row(itv, SB_E, SQ_E)
            for rr in range(L):
                i = g * L + rr
                pltpu.async_copy(ent_h.at[ihr[rr]], h_v.at[b, i], sem)
                pltpu.async_copy(relemb_h.at[irr[rr]], r_v.at[b, i], sem)
                pltpu.async_copy(ent_h.at[itr[rr]], t_v.at[b, i], sem)
            return c

        lax.fori_loop(0, CHK // L, fire, 0)

    _fire_pass(0)
    for p in range(NPASS):
        pbase = p * CHK
        b, sem = p % 2, sems[p % 2]
        if p + 1 < NPASS:
            _fire_pass(p + 1)
        pltpu.make_async_copy(ent_h.at[pl.ds(0, CHK)], h_v.at[b], sem).wait()
        pltpu.make_async_copy(ent_h.at[pl.ds(0, CHK)], r_v.at[b], sem).wait()
        pltpu.make_async_copy(ent_h.at[pl.ds(0, CHK)], t_v.at[b], sem).wait()

        def blk_body(blk, carry, pbase=pbase, b=b):
            rbase = blk * RPB
            ihv = ih_v[pl.ds(pbase + rbase, L)]
            irv = ir_v[pl.ds(pbase + rbase, L)]
            itv = it_v[pl.ds(pbase + rbase, L)]
            oh = _pack_woff(ihv, SQ_E)
            orr = _pack_woff(irv, SQ_R)
            ot = _pack_woff(itv, SQ_E)
            hh = _pack_half(ihv, SQ_E)
            hr = _pack_half(irv, SQ_R)
            ht = _pack_half(itv, SQ_E)

            def _feat(ref, row, off, shv, g, b):
                w = ref[b, row, pl.ds(off + g * L, L)]
                bits = lax.shift_left(lax.shift_right_logical(w, shv),
                                      jnp.uint32(16))
                return lax.bitcast_convert_type(bits, jnp.float32)

            outv = jnp.zeros((L,), jnp.float32)
            for rr in range(RPB):
                row = rbase + rr
                ph, pr, pt = oh[rr], orr[rr], ot[rr]
                ch = jnp.zeros((L,), jnp.uint32) + (hh[rr] * 16).astype(jnp.uint32)
                cr = jnp.zeros((L,), jnp.uint32) + (hr[rr] * 16).astype(jnp.uint32)
                ct = jnp.zeros((L,), jnp.uint32) + (ht[rr] * 16).astype(jnp.uint32)
                acc = jnp.zeros((L,), jnp.float32)
                for g in range(G):
                    acc = acc + jnp.abs(
                        _feat(h_v, row, ph, ch, g, b)
                        + _feat(r_v, row, pr, cr, g, b)
                        - _feat(t_v, row, pt, ct, g, b))
                outv = jnp.where(iot == rr, _hsum(acc), outv)
            o_v[pl.ds(pbase + rbase, RPB)] = -outv
            return carry

        lax.fori_loop(0, CHK // RPB, blk_body, 0)

    pltpu.sync_copy(o_v, out_h.at[pl.ds(wid * BPW, BPW)])


def kernel(head, relation, tail, entity_emb, relation_emb):
    ent_pk = _to_packed(entity_emb.T, NE, TCW_E)
    rel_pk = _to_packed(relation_emb.T, NR, TCW_R)
    return _transe(head.astype(jnp.int32), relation.astype(jnp.int32),
                   tail.astype(jnp.int32), ent_pk, rel_pk)
